# baseline (speedup 1.0000x reference)
import jax
import jax.numpy as jnp
from jax import lax
from jax.experimental import pallas as pl
from jax.experimental.pallas import tpu as pltpu
from jax.experimental.pallas import tpu_sc as plsc

B, H, W, C = 4, 224, 224, 96
NR = 1000
NROIS = B * NR
AH = AW = 7
NW = 32
RPW = NROIS // NW
CH = 5
NCHUNK = RPW // CH
TP = CH * AW
TPAD = 40
PIX = CH * AH * AW
OUTW = PIX * C
SCALE = 223.0
NG = C // 16


def _splat(val):
    return jnp.full((16,), val, jnp.int32)


def _body(table, roisv, out, rois_v, idx_a, xl_a, aw_a, bw_a, oidx_a, oq_a,
          dest, outb, sem0, sem1, semo):
    iota16 = lax.broadcasted_iota(jnp.int32, (16,), 0)
    wid = lax.axis_index("s") * 2 + lax.axis_index("c")
    base = wid * RPW

    pltpu.sync_copy(roisv, rois_v)

    def _zrow(j, _):
        jv = jnp.full((16,), j, jnp.int32)
        for h in range(2):
            pv = (wid * (4 * AH) + j * 2 + h) * 16 + iota16
            plsc.store_scatter(idx_a, [jv, h * TPAD + 24 + iota16], pv)
        return 0

    lax.fori_loop(0, 2 * AH, _zrow, 0)

    rr0 = jnp.minimum(iota16, CH - 1)
    for q in range(AW):
        plsc.store_scatter(oidx_a, [rr0 * AW + q], rr0)
        plsc.store_scatter(oq_a, [rr0 * AW + q], _splat(q))

    iotag = [iota16 + g * 16 for g in range(NG)]

    def gather_slices(p, buf):
        for c2 in range(2):
            yield (table.at[idx_a.at[c2 * AH + p]], dest.at[buf, c2])

    def issue(p, buf, sem):
        for src, dst in gather_slices(p, buf):
            pltpu.async_copy(src, dst, sem)

    def drain(p, buf, sem):
        for src, dst in gather_slices(p, buf):
            pltpu.make_async_copy(src, dst, sem).wait()

    def out_off(cc):
        return base + cc * CH

    def blend(p, buf):
        pofs = p * TP

        @plsc.parallel_loop(0, TP, unroll=2)
        def _px(t):
            xl = plsc.load_gather(xl_a, [_splat(t)])
            aw = plsc.load_gather(aw_a, [_splat(pofs + t)])
            bw = plsc.load_gather(bw_a, [_splat(pofs + t)])
            rv = plsc.load_gather(oidx_a, [_splat(t)])
            qv = plsc.load_gather(oq_a, [_splat(t)])
            pv = _splat(p)
            omx = 1.0 - xl
            for g in range(NG):
                sl = pl.ds(g * 16, 16)
                tlv = dest[buf, 0, t, sl]
                trv = dest[buf, 0, TPAD + t, sl]
                blv = dest[buf, 1, t, sl]
                brv = dest[buf, 1, TPAD + t, sl]
                u = tlv * omx + trv * xl
                v = blv * omx + brv * xl
                plsc.store_scatter(outb, [rv, pv, qv, iotag[g]],
                                   u * aw + v * bw)

    def chunk_body(cc, _):
        rr = jnp.minimum(iota16, CH - 1)
        rl = cc * CH + rr
        rg = base + rl
        b = ((rg >= NR).astype(jnp.int32)
             + (rg >= 2 * NR).astype(jnp.int32)
             + (rg >= 3 * NR).astype(jnp.int32))
        bb = b * (H * W)
        r4o = rg * 4
        y1 = plsc.load_gather(rois_v, [r4o])
        x1 = plsc.load_gather(rois_v, [r4o + 1])
        y2 = plsc.load_gather(rois_v, [r4o + 2])
        x2 = plsc.load_gather(rois_v, [r4o + 3])
        hs = ((y2 - y1) * SCALE) / 6.0
        ws = ((x2 - x1) * SCALE) / 6.0
        y1s = y1 * SCALE
        x1s = x1 * SCALE

        ys = []
        for p in range(AH):
            iny = y1s + float(p) * hs
            vy = (iny >= 0.0) & (iny <= SCALE)
            ty = jnp.clip(iny.astype(jnp.int32), 0, H - 2)
            yl = iny - ty.astype(jnp.float32)
            my = jnp.where(vy, 1.0, 0.0).astype(jnp.float32)
            ys.append((ty, yl * my, (1.0 - yl) * my))
        xs = []
        for q in range(AW):
            inx = x1s + float(q) * ws
            vx = (inx >= 0.0) & (inx <= SCALE)
            lx = jnp.clip(inx.astype(jnp.int32), 0, W - 2)
            xl = inx - lx.astype(jnp.float32)
            mx = jnp.where(vx, 1.0, 0.0).astype(jnp.float32)
            plsc.store_scatter(xl_a, [rr * AW + q], xl)
            xs.append((lx, mx))
        for p in range(AH):
            ty, ylm, oylm = ys[p]
            rowb = bb + ty * W
            for q in range(AW):
                lx, mx = xs[q]
                tl = rowb + lx
                toff = rr * AW + q
                plsc.store_scatter(idx_a, [_splat(p), toff], tl)
                plsc.store_scatter(idx_a, [_splat(p), TPAD + toff], tl + 1)
                plsc.store_scatter(idx_a, [_splat(AH + p), toff], tl + W)
                plsc.store_scatter(idx_a, [_splat(AH + p), TPAD + toff],
                                   tl + W + 1)
                woff = p * TP + toff
                plsc.store_scatter(aw_a, [woff], oylm * mx)
                plsc.store_scatter(bw_a, [woff], ylm * mx)

        issue(0, 0, sem0)
        for p in range(AH):
            buf = p % 2
            sem = sem0 if buf == 0 else sem1
            if p + 1 < AH:
                issue(p + 1, 1 - buf, sem1 if buf == 0 else sem0)
            drain(p, buf, sem)
            if p == 0:
                @pl.when(cc > 0)
                def _wait_out():
                    pltpu.make_async_copy(
                        outb, out.at[pl.ds(out_off(cc - 1), CH)],
                        semo).wait()
            blend(p, buf)
        pltpu.async_copy(outb, out.at[pl.ds(out_off(cc), CH)], semo)
        return 0

    lax.fori_loop(0, NCHUNK, chunk_body, 0)
    pltpu.make_async_copy(outb, out.at[pl.ds(out_off(NCHUNK - 1), CH)],
                          semo).wait()


_mesh = plsc.VectorSubcoreMesh(core_axis_name="c", subcore_axis_name="s")

_sc_call = pl.kernel(
    _body,
    out_type=jax.ShapeDtypeStruct((NROIS, AH, AW, C), jnp.float32),
    mesh=_mesh,
    compiler_params=pltpu.CompilerParams(use_tc_tiling_on_sc=False,
                                         needs_layout_passes=False),
    scratch_types=[
        pltpu.VMEM((NROIS * 4,), jnp.float32),
        pltpu.VMEM((2 * AH, 2 * TPAD), jnp.int32),
        pltpu.VMEM((TPAD,), jnp.float32),
        pltpu.VMEM((AH * TP,), jnp.float32),
        pltpu.VMEM((AH * TP,), jnp.float32),
        pltpu.VMEM((TP,), jnp.int32),
        pltpu.VMEM((TP,), jnp.int32),
        pltpu.VMEM((2, 2, 2 * TPAD, C), jnp.float32),
        pltpu.VMEM((CH, AH, AW, C), jnp.float32),
        pltpu.SemaphoreType.DMA,
        pltpu.SemaphoreType.DMA,
        pltpu.SemaphoreType.DMA,
    ],
)


def kernel(feature_map, rois):
    table = feature_map.reshape(B * H * W, C)
    return _sc_call(table, rois.reshape(NROIS * 4))


# speedup vs baseline: 1.0187x; 1.0187x over previous
import jax
import jax.numpy as jnp
from jax import lax
from jax.experimental import pallas as pl
from jax.experimental.pallas import tpu as pltpu
from jax.experimental.pallas import tpu_sc as plsc

B, H, W, C = 4, 224, 224, 96
NR = 1000
NROIS = B * NR
AH = AW = 7
NW = 32
RPW = NROIS // NW
CH = 5
NCHUNK = RPW // CH
TP = CH * AW
TPAD = 40
PIX = CH * AH * AW
OUTW = PIX * C
SCALE = 223.0
NG = C // 16


def _splat(val):
    return jnp.full((16,), val, jnp.int32)


def _body(table, roisv, out, rois_v, idx_a, xl_a, aw_a, bw_a, oidx_a, oq_a,
          dest, outb, sem0, sem1, sem2, semo):
    iota16 = lax.broadcasted_iota(jnp.int32, (16,), 0)
    wid = lax.axis_index("s") * 2 + lax.axis_index("c")
    base = wid * RPW

    pltpu.sync_copy(roisv, rois_v)

    def _zrow(j, _):
        jv = jnp.full((16,), j, jnp.int32)
        for h in range(2):
            pv = (wid * (4 * AH) + j * 2 + h) * 16 + iota16
            plsc.store_scatter(idx_a, [jv, h * TPAD + 24 + iota16], pv)
        return 0

    lax.fori_loop(0, 2 * AH, _zrow, 0)

    rr0 = jnp.minimum(iota16, CH - 1)
    for q in range(AW):
        plsc.store_scatter(oidx_a, [rr0 * AW + q], rr0)
        plsc.store_scatter(oq_a, [rr0 * AW + q], _splat(q))

    iotag = [iota16 + g * 16 for g in range(NG)]

    def gather_slices(p, buf):
        for c2 in range(2):
            yield (table.at[idx_a.at[c2 * AH + p]], dest.at[buf, c2])

    def issue(p, buf, sem):
        for src, dst in gather_slices(p, buf):
            pltpu.async_copy(src, dst, sem)

    def drain(p, buf, sem):
        for src, dst in gather_slices(p, buf):
            pltpu.make_async_copy(src, dst, sem).wait()

    def out_off(cc):
        return base + cc * CH

    def blend(p, buf):
        pofs = p * TP

        @plsc.parallel_loop(0, TP, unroll=2)
        def _px(t):
            xl = plsc.load_gather(xl_a, [_splat(t)])
            aw = plsc.load_gather(aw_a, [_splat(pofs + t)])
            bw = plsc.load_gather(bw_a, [_splat(pofs + t)])
            rv = plsc.load_gather(oidx_a, [_splat(t)])
            qv = plsc.load_gather(oq_a, [_splat(t)])
            pv = _splat(p)
            omx = 1.0 - xl
            for g in range(NG):
                sl = pl.ds(g * 16, 16)
                tlv = dest[buf, 0, t, sl]
                trv = dest[buf, 0, TPAD + t, sl]
                blv = dest[buf, 1, t, sl]
                brv = dest[buf, 1, TPAD + t, sl]
                u = tlv * omx + trv * xl
                v = blv * omx + brv * xl
                plsc.store_scatter(outb, [rv, pv, qv, iotag[g]],
                                   u * aw + v * bw)

    def chunk_body(cc, _):
        rr = jnp.minimum(iota16, CH - 1)
        rl = cc * CH + rr
        rg = base + rl
        b = ((rg >= NR).astype(jnp.int32)
             + (rg >= 2 * NR).astype(jnp.int32)
             + (rg >= 3 * NR).astype(jnp.int32))
        bb = b * (H * W)
        r4o = rg * 4
        y1 = plsc.load_gather(rois_v, [r4o])
        x1 = plsc.load_gather(rois_v, [r4o + 1])
        y2 = plsc.load_gather(rois_v, [r4o + 2])
        x2 = plsc.load_gather(rois_v, [r4o + 3])
        hs = ((y2 - y1) * SCALE) / 6.0
        ws = ((x2 - x1) * SCALE) / 6.0
        y1s = y1 * SCALE
        x1s = x1 * SCALE

        ys = []
        for p in range(AH):
            iny = y1s + float(p) * hs
            vy = (iny >= 0.0) & (iny <= SCALE)
            ty = jnp.clip(iny.astype(jnp.int32), 0, H - 2)
            yl = iny - ty.astype(jnp.float32)
            my = jnp.where(vy, 1.0, 0.0).astype(jnp.float32)
            ys.append((ty, yl * my, (1.0 - yl) * my))
        xs = []
        for q in range(AW):
            inx = x1s + float(q) * ws
            vx = (inx >= 0.0) & (inx <= SCALE)
            lx = jnp.clip(inx.astype(jnp.int32), 0, W - 2)
            xl = inx - lx.astype(jnp.float32)
            mx = jnp.where(vx, 1.0, 0.0).astype(jnp.float32)
            plsc.store_scatter(xl_a, [rr * AW + q], xl)
            xs.append((lx, mx))
        for p in range(AH):
            ty, ylm, oylm = ys[p]
            rowb = bb + ty * W
            for q in range(AW):
                lx, mx = xs[q]
                tl = rowb + lx
                toff = rr * AW + q
                plsc.store_scatter(idx_a, [_splat(p), toff], tl)
                plsc.store_scatter(idx_a, [_splat(p), TPAD + toff], tl + 1)
                plsc.store_scatter(idx_a, [_splat(AH + p), toff], tl + W)
                plsc.store_scatter(idx_a, [_splat(AH + p), TPAD + toff],
                                   tl + W + 1)
                woff = p * TP + toff
                plsc.store_scatter(aw_a, [woff], oylm * mx)
                plsc.store_scatter(bw_a, [woff], ylm * mx)

        sems = (sem0, sem1, sem2)
        issue(0, 0, sems[0])
        issue(1, 1, sems[1])
        for p in range(AH):
            buf = p % 3
            if p + 2 < AH:
                issue(p + 2, (p + 2) % 3, sems[(p + 2) % 3])
            drain(p, buf, sems[buf])
            if p == 0:
                @pl.when(cc > 0)
                def _wait_out():
                    pltpu.make_async_copy(
                        outb, out.at[pl.ds(out_off(cc - 1), CH)],
                        semo).wait()
            blend(p, buf)
        pltpu.async_copy(outb, out.at[pl.ds(out_off(cc), CH)], semo)
        return 0

    lax.fori_loop(0, NCHUNK, chunk_body, 0)
    pltpu.make_async_copy(outb, out.at[pl.ds(out_off(NCHUNK - 1), CH)],
                          semo).wait()


_mesh = plsc.VectorSubcoreMesh(core_axis_name="c", subcore_axis_name="s")

_sc_call = pl.kernel(
    _body,
    out_type=jax.ShapeDtypeStruct((NROIS, AH, AW, C), jnp.float32),
    mesh=_mesh,
    compiler_params=pltpu.CompilerParams(use_tc_tiling_on_sc=False,
                                         needs_layout_passes=False),
    scratch_types=[
        pltpu.VMEM((NROIS * 4,), jnp.float32),
        pltpu.VMEM((2 * AH, 2 * TPAD), jnp.int32),
        pltpu.VMEM((TPAD,), jnp.float32),
        pltpu.VMEM((AH * TP,), jnp.float32),
        pltpu.VMEM((AH * TP,), jnp.float32),
        pltpu.VMEM((TP,), jnp.int32),
        pltpu.VMEM((TP,), jnp.int32),
        pltpu.VMEM((3, 2, 2 * TPAD, C), jnp.float32),
        pltpu.VMEM((CH, AH, AW, C), jnp.float32),
        pltpu.SemaphoreType.DMA,
        pltpu.SemaphoreType.DMA,
        pltpu.SemaphoreType.DMA,
        pltpu.SemaphoreType.DMA,
    ],
)


def kernel(feature_map, rois):
    table = feature_map.reshape(B * H * W, C)
    return _sc_call(table, rois.reshape(NROIS * 4))


# speedup vs baseline: 1.0911x; 1.0710x over previous
import jax
import jax.numpy as jnp
from jax import lax
from jax.experimental import pallas as pl
from jax.experimental.pallas import tpu as pltpu
from jax.experimental.pallas import tpu_sc as plsc

B, H, W, C = 4, 224, 224, 96
NR = 1000
NROIS = B * NR
AH = AW = 7
NW = 32
CH = 8
NW_BIG = 20
TP = CH * AW
TPAD = TP
PIX = CH * AH * AW
OUTW = PIX * C
SCALE = 223.0
NG = C // 16


def _splat(val):
    return jnp.full((16,), val, jnp.int32)


def _body(table, roisv, out, rois_v, idx_a, xl_a, aw_a, bw_a, oidx_a, oq_a,
          dest, outb, sem0, sem1, sem2, semo):
    iota16 = lax.broadcasted_iota(jnp.int32, (16,), 0)
    wid = lax.axis_index("s") * 2 + lax.axis_index("c")
    big = wid < NW_BIG
    base = jnp.where(big, wid * (16 * CH),
                     NW_BIG * (16 * CH) + (wid - NW_BIG) * (15 * CH))
    nchunk = jnp.where(big, 16, 15)

    wstart = jnp.minimum(base, NROIS - 16 * CH)
    loff = base - wstart
    pltpu.sync_copy(roisv.at[pl.ds(wstart * 4, 16 * CH * 4)], rois_v)

    rr0 = jnp.minimum(iota16, CH - 1)
    for q in range(AW):
        plsc.store_scatter(oidx_a, [rr0 * AW + q], rr0)
        plsc.store_scatter(oq_a, [rr0 * AW + q], _splat(q))

    iotag = [iota16 + g * 16 for g in range(NG)]

    def gather_slices(p, buf):
        for c2 in range(2):
            yield (table.at[idx_a.at[c2 * AH + p]], dest.at[buf, c2])

    def issue(p, buf, sem):
        for src, dst in gather_slices(p, buf):
            pltpu.async_copy(src, dst, sem)

    def drain(p, buf, sem):
        for src, dst in gather_slices(p, buf):
            pltpu.make_async_copy(src, dst, sem).wait()

    def out_off(cc):
        return base + cc * CH

    def blend(p, buf):
        pofs = p * TP

        @plsc.parallel_loop(0, TP, unroll=2)
        def _px(t):
            xl = plsc.load_gather(xl_a, [_splat(t)])
            aw = plsc.load_gather(aw_a, [_splat(pofs + t)])
            bw = plsc.load_gather(bw_a, [_splat(pofs + t)])
            rv = plsc.load_gather(oidx_a, [_splat(t)])
            qv = plsc.load_gather(oq_a, [_splat(t)])
            pv = _splat(p)
            omx = 1.0 - xl
            for g in range(NG):
                sl = pl.ds(g * 16, 16)
                tlv = dest[buf, 0, t, sl]
                trv = dest[buf, 0, TPAD + t, sl]
                blv = dest[buf, 1, t, sl]
                brv = dest[buf, 1, TPAD + t, sl]
                u = tlv * omx + trv * xl
                v = blv * omx + brv * xl
                plsc.store_scatter(outb, [rv, pv, qv, iotag[g]],
                                   u * aw + v * bw)

    def chunk_body(cc, _):
        rr = jnp.minimum(iota16, CH - 1)
        rl = cc * CH + rr
        rg = base + rl
        b = ((rg >= NR).astype(jnp.int32)
             + (rg >= 2 * NR).astype(jnp.int32)
             + (rg >= 3 * NR).astype(jnp.int32))
        bb = b * (H * W)
        r4o = (rl + loff) * 4
        y1 = plsc.load_gather(rois_v, [r4o])
        x1 = plsc.load_gather(rois_v, [r4o + 1])
        y2 = plsc.load_gather(rois_v, [r4o + 2])
        x2 = plsc.load_gather(rois_v, [r4o + 3])
        hs = ((y2 - y1) * SCALE) / 6.0
        ws = ((x2 - x1) * SCALE) / 6.0
        y1s = y1 * SCALE
        x1s = x1 * SCALE

        ys = []
        for p in range(AH):
            iny = y1s + float(p) * hs
            vy = (iny >= 0.0) & (iny <= SCALE)
            ty = jnp.clip(iny.astype(jnp.int32), 0, H - 2)
            yl = iny - ty.astype(jnp.float32)
            my = jnp.where(vy, 1.0, 0.0).astype(jnp.float32)
            ys.append((ty, yl * my, (1.0 - yl) * my))
        xs = []
        for q in range(AW):
            inx = x1s + float(q) * ws
            vx = (inx >= 0.0) & (inx <= SCALE)
            lx = jnp.clip(inx.astype(jnp.int32), 0, W - 2)
            xl = inx - lx.astype(jnp.float32)
            mx = jnp.where(vx, 1.0, 0.0).astype(jnp.float32)
            plsc.store_scatter(xl_a, [rr * AW + q], xl)
            xs.append((lx, mx))
        for p in range(AH):
            ty, ylm, oylm = ys[p]
            rowb = bb + ty * W
            for q in range(AW):
                lx, mx = xs[q]
                tl = rowb + lx
                toff = rr * AW + q
                plsc.store_scatter(idx_a, [_splat(p), toff], tl)
                plsc.store_scatter(idx_a, [_splat(p), TPAD + toff], tl + 1)
                plsc.store_scatter(idx_a, [_splat(AH + p), toff], tl + W)
                plsc.store_scatter(idx_a, [_splat(AH + p), TPAD + toff],
                                   tl + W + 1)
                woff = p * TP + toff
                plsc.store_scatter(aw_a, [woff], oylm * mx)
                plsc.store_scatter(bw_a, [woff], ylm * mx)

        sems = (sem0, sem1, sem2)
        issue(0, 0, sems[0])
        issue(1, 1, sems[1])
        for p in range(AH):
            buf = p % 3
            if p + 2 < AH:
                issue(p + 2, (p + 2) % 3, sems[(p + 2) % 3])
            drain(p, buf, sems[buf])
            if p == 0:
                @pl.when(cc > 0)
                def _wait_out():
                    pltpu.make_async_copy(
                        outb, out.at[pl.ds(out_off(cc - 1), CH)],
                        semo).wait()
            blend(p, buf)
        pltpu.async_copy(outb, out.at[pl.ds(out_off(cc), CH)], semo)
        return 0

    lax.fori_loop(0, nchunk, chunk_body, 0)
    pltpu.make_async_copy(outb, out.at[pl.ds(out_off(nchunk - 1), CH)],
                          semo).wait()


_mesh = plsc.VectorSubcoreMesh(core_axis_name="c", subcore_axis_name="s")

_sc_call = pl.kernel(
    _body,
    out_type=jax.ShapeDtypeStruct((NROIS, AH, AW, C), jnp.float32),
    mesh=_mesh,
    compiler_params=pltpu.CompilerParams(use_tc_tiling_on_sc=False,
                                         needs_layout_passes=False),
    scratch_types=[
        pltpu.VMEM((16 * CH * 4,), jnp.float32),
        pltpu.VMEM((2 * AH, 2 * TPAD), jnp.int32),
        pltpu.VMEM((TPAD,), jnp.float32),
        pltpu.VMEM((AH * TP,), jnp.float32),
        pltpu.VMEM((AH * TP,), jnp.float32),
        pltpu.VMEM((TP,), jnp.int32),
        pltpu.VMEM((TP,), jnp.int32),
        pltpu.VMEM((3, 2, 2 * TPAD, C), jnp.float32),
        pltpu.VMEM((CH, AH, AW, C), jnp.float32),
        pltpu.SemaphoreType.DMA,
        pltpu.SemaphoreType.DMA,
        pltpu.SemaphoreType.DMA,
        pltpu.SemaphoreType.DMA,
    ],
)


def kernel(feature_map, rois):
    table = feature_map.reshape(B * H * W, C)
    return _sc_call(table, rois.reshape(NROIS * 4))


# speedup vs baseline: 1.0920x; 1.0009x over previous
import jax
import jax.numpy as jnp
from jax import lax
from jax.experimental import pallas as pl
from jax.experimental.pallas import tpu as pltpu
from jax.experimental.pallas import tpu_sc as plsc

B, H, W, C = 4, 224, 224, 96
NR = 1000
NROIS = B * NR
AH = AW = 7
NW = 32
CH = 8
NW_BIG = 20
TP = CH * AW
TPAD = TP
SCALE = 223.0
NG = C // 16


def _splat(val):
    return jnp.full((16,), val, jnp.int32)


def _body(table, roisv, out, rois_v, idx_a, xl_a, aw_a, bw_a, oidx_a, oq_a,
          dest, outb, sem0, sem1, sem2, semo):
    iota16 = lax.broadcasted_iota(jnp.int32, (16,), 0)
    wid = lax.axis_index("s") * 2 + lax.axis_index("c")
    big = wid < NW_BIG
    base = jnp.where(big, wid * (16 * CH),
                     NW_BIG * (16 * CH) + (wid - NW_BIG) * (15 * CH))
    nchunk = jnp.where(big, 16, 15)

    wstart = jnp.minimum(base, NROIS - 16 * CH)
    loff = base - wstart
    pltpu.sync_copy(roisv.at[pl.ds(wstart * 4, 16 * CH * 4)], rois_v)

    rr0 = jnp.minimum(iota16, CH - 1)
    for q in range(AW):
        plsc.store_scatter(oidx_a, [rr0 * AW + q], rr0)
        plsc.store_scatter(oq_a, [rr0 * AW + q], _splat(q))

    iotag = [iota16 + g * 16 for g in range(NG)]

    def gather_slices(p, buf):
        for c2 in range(2):
            yield (table.at[idx_a.at[c2 * AH + p]], dest.at[buf, c2])

    def issue(p, buf, sem):
        for src, dst in gather_slices(p, buf):
            pltpu.async_copy(src, dst, sem)

    def drain(p, buf, sem):
        for src, dst in gather_slices(p, buf):
            pltpu.make_async_copy(src, dst, sem).wait()

    def out_off(cc):
        return base + cc * CH

    def blend(p, buf):
        pofs = p * TP

        @plsc.parallel_loop(0, TP, unroll=2)
        def _px(t):
            xl = plsc.load_gather(xl_a, [_splat(t)])
            aw = plsc.load_gather(aw_a, [_splat(pofs + t)])
            bw = plsc.load_gather(bw_a, [_splat(pofs + t)])
            rv = plsc.load_gather(oidx_a, [_splat(t)])
            qv = plsc.load_gather(oq_a, [_splat(t)])
            pv = _splat(p)
            omx = 1.0 - xl
            for g in range(NG):
                sl = pl.ds(g * 16, 16)
                tlv = dest[buf, 0, t, sl]
                trv = dest[buf, 0, TPAD + t, sl]
                blv = dest[buf, 1, t, sl]
                brv = dest[buf, 1, TPAD + t, sl]
                u = tlv * omx + trv * xl
                v = blv * omx + brv * xl
                plsc.store_scatter(outb, [rv, pv, qv, iotag[g]],
                                   u * aw + v * bw)

    def chunk_body(cc, _):
        rr = jnp.minimum(iota16, CH - 1)
        rl = cc * CH + rr
        rg = base + rl
        b = ((rg >= NR).astype(jnp.int32)
             + (rg >= 2 * NR).astype(jnp.int32)
             + (rg >= 3 * NR).astype(jnp.int32))
        bb = b * (H * W)
        r4o = (rl + loff) * 4
        y1 = plsc.load_gather(rois_v, [r4o])
        x1 = plsc.load_gather(rois_v, [r4o + 1])
        y2 = plsc.load_gather(rois_v, [r4o + 2])
        x2 = plsc.load_gather(rois_v, [r4o + 3])
        hs = ((y2 - y1) * SCALE) / 6.0
        ws = ((x2 - x1) * SCALE) / 6.0
        y1s = y1 * SCALE
        x1s = x1 * SCALE

        ys = []
        for p in range(AH):
            iny = y1s + float(p) * hs
            vy = (iny >= 0.0) & (iny <= SCALE)
            ty = jnp.clip(iny.astype(jnp.int32), 0, H - 2)
            yl = iny - ty.astype(jnp.float32)
            my = jnp.where(vy, 1.0, 0.0).astype(jnp.float32)
            ys.append((ty, yl * my, (1.0 - yl) * my))
        xs = []
        for q in range(AW):
            inx = x1s + float(q) * ws
            vx = (inx >= 0.0) & (inx <= SCALE)
            lx = jnp.clip(inx.astype(jnp.int32), 0, W - 2)
            xl = inx - lx.astype(jnp.float32)
            mx = jnp.where(vx, 1.0, 0.0).astype(jnp.float32)
            plsc.store_scatter(xl_a, [rr * AW + q], xl)
            xs.append((lx, mx))
        for p in range(AH):
            ty, ylm, oylm = ys[p]
            rowb = bb + ty * W
            for q in range(AW):
                lx, mx = xs[q]
                tl = rowb + lx
                toff = rr * AW + q
                plsc.store_scatter(idx_a, [_splat(p), toff], tl)
                plsc.store_scatter(idx_a, [_splat(p), TPAD + toff], tl + 1)
                plsc.store_scatter(idx_a, [_splat(AH + p), toff], tl + W)
                plsc.store_scatter(idx_a, [_splat(AH + p), TPAD + toff],
                                   tl + W + 1)
                woff = p * TP + toff
                plsc.store_scatter(aw_a, [woff], oylm * mx)
                plsc.store_scatter(bw_a, [woff], ylm * mx)

        sems = (sem0, sem1, sem2)
        issue(0, 0, sems[0])
        issue(1, 1, sems[1])
        for p in range(AH):
            buf = p % 3
            if p + 2 < AH:
                issue(p + 2, (p + 2) % 3, sems[(p + 2) % 3])
            drain(p, buf, sems[buf])
            if p == 0:
                @pl.when(cc > 0)
                def _wait_out():
                    pltpu.make_async_copy(
                        outb, out.at[pl.ds(out_off(cc - 1), CH)],
                        semo).wait()
            blend(p, buf)
        pltpu.async_copy(outb, out.at[pl.ds(out_off(cc), CH)], semo)
        return 0

    lax.fori_loop(0, nchunk, chunk_body, 0)
    pltpu.make_async_copy(outb, out.at[pl.ds(out_off(nchunk - 1), CH)],
                          semo).wait()


_mesh = plsc.VectorSubcoreMesh(core_axis_name="c", subcore_axis_name="s")

_sc_call = pl.kernel(
    _body,
    out_type=jax.ShapeDtypeStruct((NROIS, AH, AW, C), jnp.float32),
    mesh=_mesh,
    compiler_params=pltpu.CompilerParams(use_tc_tiling_on_sc=False,
                                         needs_layout_passes=False),
    scratch_types=[
        pltpu.VMEM((16 * CH * 4,), jnp.float32),
        pltpu.VMEM((2 * AH, 2 * TPAD), jnp.int32),
        pltpu.VMEM((TPAD,), jnp.float32),
        pltpu.VMEM((AH * TP,), jnp.float32),
        pltpu.VMEM((AH * TP,), jnp.float32),
        pltpu.VMEM((TP,), jnp.int32),
        pltpu.VMEM((TP,), jnp.int32),
        pltpu.VMEM((3, 2, 2 * TPAD, C), jnp.float32),
        pltpu.VMEM((CH, AH, AW, C), jnp.float32),
        pltpu.SemaphoreType.DMA,
        pltpu.SemaphoreType.DMA,
        pltpu.SemaphoreType.DMA,
        pltpu.SemaphoreType.DMA,
    ],
)


def kernel(feature_map, rois):
    table = feature_map.reshape(B * H * W, C)
    return _sc_call(table, rois.reshape(NROIS * 4))
